# baseline (speedup 1.0000x reference)
import functools

import jax
import jax.numpy as jnp
from jax import lax
from jax.experimental import pallas as pl
from jax.experimental.pallas import tpu as pltpu
from jax.experimental.pallas import tpu_sc as plsc

N_NODES = 10000
N_EDGES = 320000
D_EDGE = 16
CHUNK = 128
N_CHUNKS = N_EDGES // CHUNK
HALVES = 2
CPH = N_CHUNKS // HALVES
CBLK = 125
NBLK = CPH // CBLK
LANES = 16


def _sc_segment_sum(e4, ei3):
    mesh = plsc.VectorSubcoreMesh(core_axis_name="c", subcore_axis_name="s")

    @functools.partial(
        pl.kernel,
        mesh=mesh,
        out_type=jax.ShapeDtypeStruct((D_EDGE, N_NODES), jnp.float32),
        compiler_params=pltpu.CompilerParams(use_tc_tiling_on_sc=False,
                                             needs_layout_passes=False),
        scratch_types=[
            pltpu.VMEM((CBLK, 1, CHUNK), jnp.int32),
            pltpu.VMEM((CBLK, 1, CHUNK), jnp.int32),
            pltpu.VMEM((CBLK, 1, CHUNK), jnp.float32),
            pltpu.VMEM((CBLK, 1, CHUNK), jnp.float32),
            pltpu.VMEM((N_NODES,), jnp.float32),
            pltpu.VMEM((N_NODES,), jnp.float32),
            pltpu.VMEM_SHARED((D_EDGE, N_NODES), jnp.float32),
            pltpu.SemaphoreType.DMA,
            pltpu.SemaphoreType.DMA,
        ],
    )
    def k(e_hbm, ei_hbm, out_hbm,
          idx_v0, idx_v1, val_v0, val_v1, acc_v, part_v, sp, sem0, sem1):
        cid = lax.axis_index("c")
        sid = lax.axis_index("s")
        f = (sid % 8) * 2 + cid
        h = sid // 8
        g = f // 8
        r = f % 8

        idx_bufs = (idx_v0, idx_v1)
        val_bufs = (val_v0, val_v1)
        sems = (sem0, sem1)

        def zero_step(i, carry):
            acc_v[pl.ds(i * LANES, LANES)] = jnp.zeros((LANES,), jnp.float32)
            return carry

        lax.fori_loop(0, N_NODES // LANES, zero_step, 0)

        def start(b):
            c0 = h * CPH + b * CBLK
            i = b % 2
            cp1 = pltpu.async_copy(
                ei_hbm.at[pl.ds(c0, CBLK), pl.ds(1, 1)], idx_bufs[i], sems[i])
            cp2 = pltpu.async_copy(
                e_hbm.at[g, pl.ds(c0, CBLK), pl.ds(r, 1)], val_bufs[i],
                sems[i])
            return cp1, cp2

        cps = start(0)
        for b in range(NBLK):
            cp1, cp2 = cps
            cp1.wait()
            cp2.wait()
            if b + 1 < NBLK:
                cps = start(b + 1)
            iv = idx_bufs[b % 2]
            vv = val_bufs[b % 2]

            def chunk_step(c4, carry2):
                pairs = []
                for u in range(4):
                    c = c4 * 4 + u
                    for t in range(CHUNK // LANES):
                        i16 = iv[c, 0, pl.ds(t * LANES, LANES)]
                        v16 = vv[c, 0, pl.ds(t * LANES, LANES)]
                        pairs.append((i16, v16))
                for i16, v16 in pairs:
                    plsc.addupdate_scatter(acc_v, [i16], v16)
                return carry2

            lax.fori_loop(0, CBLK // 4, chunk_step, 0)
            cl = CBLK - 1
            for t in range(CHUNK // LANES):
                i16 = iv[cl, 0, pl.ds(t * LANES, LANES)]
                v16 = vv[cl, 0, pl.ds(t * LANES, LANES)]
                plsc.addupdate_scatter(acc_v, [i16], v16)

        @pl.when(h == 1)
        def _():
            pltpu.sync_copy(acc_v, sp.at[f])

        plsc.subcore_barrier()

        @pl.when(h == 0)
        def _():
            pltpu.sync_copy(sp.at[f], part_v)

            def add_step(i, carry):
                plsc.addupdate(acc_v.at[pl.ds(i * LANES, LANES)],
                               part_v[pl.ds(i * LANES, LANES)])
                return carry

            lax.fori_loop(0, N_NODES // LANES, add_step, 0)
            pltpu.sync_copy(acc_v, out_hbm.at[f])

    return k(e4, ei3)


def kernel(h, edge_index, e):
    del h
    ei3 = edge_index.astype(jnp.int32).reshape(2, N_CHUNKS, CHUNK).transpose(1, 0, 2)
    e4 = e.T.reshape(2, 8, N_CHUNKS, CHUNK).transpose(0, 2, 1, 3)
    out_t = _sc_segment_sum(e4, ei3)
    return out_t.T
